# baseline (speedup 1.0000x reference)
import functools

import jax
import jax.numpy as jnp
import numpy as np
from jax import lax
from jax.experimental import pallas as pl
from jax.experimental.pallas import tpu as pltpu
from jax.experimental.pallas import tpu_sc as plsc

N = 10000
E = 160000
D = 16
P = 8
NC = 2
NS = 16
NW = NC * NS
BPW = E // NW
BE = 8000
BP = BE // P
EP = E // P


def _sc_gather(table, idx):
    mesh = plsc.VectorSubcoreMesh(core_axis_name="c", subcore_axis_name="s")

    @functools.partial(
        pl.kernel,
        mesh=mesh,
        out_type=jax.ShapeDtypeStruct((E, D), jnp.float32),
        compiler_params=pltpu.CompilerParams(use_tc_tiling_on_sc=False),
        scratch_types=[
            pltpu.VMEM((BPW,), jnp.int32),
            pltpu.VMEM((BPW, D), jnp.float32),
            pltpu.SemaphoreType.DMA,
        ],
    )
    def gather_k(table_hbm, idx_hbm, out_hbm, idx_v, rows_v, sem):
        wid = lax.axis_index("s") * NC + lax.axis_index("c")
        base = wid * BPW
        pltpu.sync_copy(idx_hbm.at[pl.ds(base, BPW)], idx_v)
        pltpu.async_copy(table_hbm.at[idx_v], rows_v, sem).wait()
        pltpu.sync_copy(rows_v, out_hbm.at[pl.ds(base, BPW)])

    return gather_k(table, idx)


def _sc_gather_half(table, idx):
    mesh = plsc.VectorSubcoreMesh(core_axis_name="c", subcore_axis_name="s")
    bs, bl = 2496, 2624

    @functools.partial(
        pl.kernel,
        mesh=mesh,
        out_type=jax.ShapeDtypeStruct((E // 2, D), jnp.float32),
        compiler_params=pltpu.CompilerParams(use_tc_tiling_on_sc=False),
        scratch_types=[
            pltpu.VMEM((bl,), jnp.int32),
            pltpu.VMEM((bl, D), jnp.float32),
            pltpu.SemaphoreType.DMA,
        ],
    )
    def gather_k(table_hbm, idx_hbm, out_hbm, idx_v, rows_v, sem):
        wid = lax.axis_index("s") * NC + lax.axis_index("c")
        base = wid * bs
        pltpu.sync_copy(idx_hbm.at[pl.ds(base, bl)], idx_v)
        pltpu.async_copy(table_hbm.at[idx_v], rows_v, sem).wait()
        pltpu.sync_copy(rows_v, out_hbm.at[pl.ds(base, bl)])

    return gather_k(table, idx)


_TEX = np.tile(np.eye(D, dtype=np.float32), (1, D))
_SEL = np.kron(np.eye(D, dtype=np.float32), np.ones((D, 1), np.float32))


def _dot(a, b):
    return jnp.dot(a, b, preferred_element_type=jnp.float32)


def _ln_relu(x, g, be, mbd):
    mu = _dot(x, mbd)
    xc = x - mu
    var = _dot(xc * xc, mbd)
    return jnp.maximum(xc * lax.rsqrt(var + 1e-5) * g + be, 0.0)


def _tc_body(ewp, erp, bsp, hsp, w1bd, w1bspr, b1t, g1t, be1t, w2bd, b2t,
             g2t, be2t, w3bd, b3t, mbd, spr, tex8, sel8, out):
    x = _dot(ewp[...], w1bd[...]) + _dot(erp[...], w1bspr[...]) + b1t[...]
    x = _ln_relu(x, g1t[...], be1t[...], mbd[...])
    x = _dot(x, w2bd[...]) + b2t[...]
    x = _ln_relu(x, g2t[...], be2t[...], mbd[...])
    bf = jnp.bfloat16
    r = (_dot(x, w3bd[...]) + b3t[...]).astype(bf)

    bh = hsp[...] * _dot(bsp[...], spr[...])
    hbig = _dot(bh, tex8[...]).astype(bf)
    out[...] = _dot(r * hbig, sel8[...])


def _tc_conv(ewp, erp, bsp, hsp, w1a, w1b, b1, g1, be1, w2, b2, g2, be2, w3,
             b3):
    f32 = jnp.float32
    eye8 = jnp.eye(P, dtype=f32)
    kr = lambda w: jnp.kron(eye8, w)
    tile8 = lambda v: jnp.tile(v.reshape(1, -1), (1, P))
    mbd = kr(jnp.full((32, 32), 1.0 / 32.0, f32))
    args = (
        ewp, erp, bsp, hsp,
        kr(w1a),
        kr(w1b.reshape(1, 32)),
        tile8(b1), tile8(g1), tile8(be1),
        kr(w2),
        tile8(b2), tile8(g2), tile8(be2),
        kr(w3),
        tile8(b3),
        mbd,
        kr(jnp.ones((1, D), f32)),
        kr(jnp.asarray(_TEX)),
        kr(jnp.asarray(_SEL)).astype(jnp.bfloat16),
    )
    edge_spec = lambda w: pl.BlockSpec((BP, w), lambda i: (i, 0))
    full = lambda a: pl.BlockSpec(a.shape, lambda i: (0,) * a.ndim)
    in_specs = [edge_spec(128), edge_spec(P), edge_spec(P), edge_spec(128)]
    in_specs += [full(a) for a in args[4:]]
    nblk = ewp.shape[0] // BP
    return pl.pallas_call(
        _tc_body,
        grid=(nblk,),
        in_specs=in_specs,
        out_specs=pl.BlockSpec((BP, 128), lambda i: (i, 0)),
        out_shape=jax.ShapeDtypeStruct((ewp.shape[0], 128), f32),
    )(*args)


def kernel(h_0, edge_index, edge_r, edge_w, basis_00, W1, b1, g1, be1, W2,
           b2, g2, be2, W3, b3):
    table = h_0.reshape(N, D)
    idx = edge_index[0].astype(jnp.int32)
    EH, EPH = E // 2, EP // 2
    h_a = _sc_gather_half(table, idx[:EH])
    h_b = _sc_gather_half(table, idx[EH:])
    ewp = edge_w.reshape(EP, 128)
    erp = edge_r.reshape(EP, P)
    bsp = basis_00.reshape(EP, P)
    w = (W1[:D], W1[D:], b1, g1, be1, W2, b2, g2, be2, W3, b3)
    out_a = _tc_conv(ewp[:EPH], erp[:EPH], bsp[:EPH],
                     h_a.reshape(EPH, 128), *w)
    out_b = _tc_conv(ewp[EPH:], erp[EPH:], bsp[EPH:],
                     h_b.reshape(EPH, 128), *w)
    return jnp.concatenate([out_a, out_b], axis=0).reshape(E, D, 1)


# speedup vs baseline: 1.1877x; 1.1877x over previous
import functools

import jax
import jax.numpy as jnp
import numpy as np
from jax import lax
from jax.experimental import pallas as pl
from jax.experimental.pallas import tpu as pltpu
from jax.experimental.pallas import tpu_sc as plsc

N = 10000
E = 160000
D = 16
P = 8
NC = 2
NS = 16
NW = NC * NS
BPW = E // NW
BE = 8000
BP = BE // P
EP = E // P


def _sc_gather(table, idx):
    mesh = plsc.VectorSubcoreMesh(core_axis_name="c", subcore_axis_name="s")

    @functools.partial(
        pl.kernel,
        mesh=mesh,
        out_type=jax.ShapeDtypeStruct((E, D), jnp.float32),
        compiler_params=pltpu.CompilerParams(use_tc_tiling_on_sc=False),
        scratch_types=[
            pltpu.VMEM((BPW,), jnp.int32),
            pltpu.VMEM((BPW, D), jnp.float32),
            pltpu.SemaphoreType.DMA,
        ],
    )
    def gather_k(table_hbm, idx_hbm, out_hbm, idx_v, rows_v, sem):
        wid = lax.axis_index("s") * NC + lax.axis_index("c")
        base = wid * BPW
        pltpu.sync_copy(idx_hbm.at[pl.ds(base, BPW)], idx_v)
        pltpu.async_copy(table_hbm.at[idx_v], rows_v, sem).wait()
        pltpu.sync_copy(rows_v, out_hbm.at[pl.ds(base, BPW)])

    return gather_k(table, idx)


_TEX = np.tile(np.eye(D, dtype=np.float32), (1, D))
_SEL = np.kron(np.eye(D, dtype=np.float32), np.ones((D, 1), np.float32))


def _dot(a, b):
    return jnp.dot(a, b, preferred_element_type=jnp.float32)


def _ln_relu(x, g, be, mbd):
    mu = _dot(x, mbd)
    xc = x - mu
    var = _dot(xc * xc, mbd)
    return jnp.maximum(xc * lax.rsqrt(var + 1e-5) * g + be, 0.0)


def _tc_body(ewp, erp, bsp, hsp, w1bd, w1bspr, b1t, g1t, be1t, w2bd, b2t,
             g2t, be2t, w3bd, b3t, mbd, spr, tex8, sel8, out):
    bfc = lambda a: a.astype(jnp.bfloat16)
    x = _dot(ewp[...], w1bd[...]) + _dot(erp[...], w1bspr[...]) + b1t[...]
    x = bfc(_ln_relu(x, g1t[...], be1t[...], mbd[...]))
    x = _dot(x, w2bd[...]) + b2t[...]
    x = bfc(_ln_relu(x, g2t[...], be2t[...], mbd[...]))
    bf = jnp.bfloat16
    r = (_dot(x, w3bd[...]) + b3t[...]).astype(bf)

    bh = hsp[...] * _dot(bsp[...], spr[...])
    hbig = _dot(bh, tex8[...]).astype(bf)
    out[...] = _dot(r * hbig, sel8[...])


def _tc_conv(ewp, erp, bsp, hsp, w1a, w1b, b1, g1, be1, w2, b2, g2, be2, w3,
             b3):
    f32 = jnp.float32
    eye8 = jnp.eye(P, dtype=f32)
    kr = lambda w: jnp.kron(eye8, w)
    tile8 = lambda v: jnp.tile(v.reshape(1, -1), (1, P))
    mbd = kr(jnp.full((32, 32), 1.0 / 32.0, f32))
    args = (
        ewp, erp, bsp, hsp,
        kr(w1a),
        kr(w1b.reshape(1, 32)),
        tile8(b1), tile8(g1), tile8(be1),
        kr(w2),
        tile8(b2), tile8(g2), tile8(be2),
        kr(w3),
        tile8(b3),
        mbd,
        kr(jnp.ones((1, D), f32)),
        kr(jnp.asarray(_TEX)),
        kr(jnp.asarray(_SEL)).astype(jnp.bfloat16),
    )
    edge_spec = lambda w: pl.BlockSpec((BP, w), lambda i: (i, 0))
    full = lambda a: pl.BlockSpec(a.shape, lambda i: (0,) * a.ndim)
    in_specs = [edge_spec(128), edge_spec(P), edge_spec(P), edge_spec(128)]
    in_specs += [full(a) for a in args[4:]]
    return pl.pallas_call(
        _tc_body,
        grid=(EP // BP,),
        in_specs=in_specs,
        out_specs=pl.BlockSpec((BP, 128), lambda i: (i, 0)),
        out_shape=jax.ShapeDtypeStruct((EP, 128), f32),
    )(*args)


def kernel(h_0, edge_index, edge_r, edge_w, basis_00, W1, b1, g1, be1, W2,
           b2, g2, be2, W3, b3):
    table = h_0.reshape(N, D)
    idx = edge_index[0].astype(jnp.int32)
    h_src = _sc_gather(table, idx)

    out_p = _tc_conv(
        edge_w.reshape(EP, 128), edge_r.reshape(EP, P),
        basis_00.reshape(EP, P), h_src.reshape(EP, 128),
        W1[:D], W1[D:], b1, g1, be1, W2, b2, g2, be2, W3, b3)
    return out_p.reshape(E, D, 1)


# speedup vs baseline: 1.3128x; 1.1053x over previous
import functools

import jax
import jax.numpy as jnp
import numpy as np
from jax import lax
from jax.experimental import pallas as pl
from jax.experimental.pallas import tpu as pltpu
from jax.experimental.pallas import tpu_sc as plsc

N = 10000
E = 160000
D = 16
P = 8
NC = 2
NS = 16
NW = NC * NS
BPW = E // NW
BE = 8000
BP = BE // P
EP = E // P


def _sc_gather(table, idx):
    mesh = plsc.VectorSubcoreMesh(core_axis_name="c", subcore_axis_name="s")

    @functools.partial(
        pl.kernel,
        mesh=mesh,
        out_type=jax.ShapeDtypeStruct((E, D), jnp.float32),
        compiler_params=pltpu.CompilerParams(use_tc_tiling_on_sc=False),
        scratch_types=[
            pltpu.VMEM((BPW,), jnp.int32),
            pltpu.VMEM((BPW, D), jnp.float32),
            pltpu.SemaphoreType.DMA,
        ],
    )
    def gather_k(table_hbm, idx_hbm, out_hbm, idx_v, rows_v, sem):
        wid = lax.axis_index("s") * NC + lax.axis_index("c")
        base = wid * BPW
        pltpu.sync_copy(idx_hbm.at[pl.ds(base, BPW)], idx_v)
        pltpu.async_copy(table_hbm.at[idx_v], rows_v, sem).wait()
        pltpu.sync_copy(rows_v, out_hbm.at[pl.ds(base, BPW)])

    return gather_k(table, idx)


_TEX = np.tile(np.eye(D, dtype=np.float32), (1, D))
_SEL = np.kron(np.eye(D, dtype=np.float32), np.ones((D, 1), np.float32))


def _dot(a, b):
    return jnp.dot(a, b, preferred_element_type=jnp.float32)


def _ln_relu(x, g, be, mbd):
    mu = _dot(x, mbd)
    xc = x - mu
    var = _dot(xc * xc, mbd)
    return jnp.maximum(xc * lax.rsqrt(var + 1e-5) * g + be, 0.0)


def _tc_body(ewp, erp, bsp, hsp, w1bd, w1bspr, b1t, g1t, be1t, w2bd, b2t,
             g2t, be2t, w3bd, b3t, mbd, spr, tex8, sel8, out):
    x = _dot(ewp[...], w1bd[...]) + _dot(erp[...], w1bspr[...]) + b1t[...]
    x = _ln_relu(x, g1t[...], be1t[...], mbd[...])
    x = _dot(x, w2bd[...]) + b2t[...]
    x = _ln_relu(x, g2t[...], be2t[...], mbd[...])
    bf = jnp.bfloat16
    r = (_dot(x, w3bd[...]) + b3t[...]).astype(bf)

    bh = hsp[...] * _dot(bsp[...], spr[...])
    hbig = _dot(bh, tex8[...]).astype(bf)
    out[...] = _dot(r * hbig, sel8[...])


def _tc_conv(ewp, erp, bsp, hsp, w1a, w1b, b1, g1, be1, w2, b2, g2, be2, w3,
             b3):
    f32 = jnp.float32
    eye8 = jnp.eye(P, dtype=f32)
    kr = lambda w: jnp.kron(eye8, w)
    tile8 = lambda v: jnp.tile(v.reshape(1, -1), (1, P))
    mbd = kr(jnp.full((32, 32), 1.0 / 32.0, f32))
    args = (
        ewp, erp, bsp, hsp,
        kr(w1a),
        kr(w1b.reshape(1, 32)),
        tile8(b1), tile8(g1), tile8(be1),
        kr(w2),
        tile8(b2), tile8(g2), tile8(be2),
        kr(w3),
        tile8(b3),
        mbd,
        kr(jnp.ones((1, D), f32)),
        kr(jnp.asarray(_TEX)),
        kr(jnp.asarray(_SEL)).astype(jnp.bfloat16),
    )
    edge_spec = lambda w: pl.BlockSpec((BP, w), lambda i: (i, 0))
    full = lambda a: pl.BlockSpec(a.shape, lambda i: (0,) * a.ndim)
    in_specs = [edge_spec(128), edge_spec(P), edge_spec(P), edge_spec(128)]
    in_specs += [full(a) for a in args[4:]]
    return pl.pallas_call(
        _tc_body,
        grid=(EP // BP,),
        in_specs=in_specs,
        out_specs=pl.BlockSpec((BP, 128), lambda i: (i, 0)),
        out_shape=jax.ShapeDtypeStruct((EP, 128), f32),
    )(*args)


def kernel(h_0, edge_index, edge_r, edge_w, basis_00, W1, b1, g1, be1, W2,
           b2, g2, be2, W3, b3):
    table = h_0.reshape(N, D)
    idx = edge_index[0].astype(jnp.int32)
    h_src = _sc_gather(table, idx)

    out_p = _tc_conv(
        edge_w.reshape(EP, 128), edge_r.reshape(EP, P),
        basis_00.reshape(EP, P), h_src.reshape(EP, 128),
        W1[:D], W1[D:], b1, g1, be1, W2, b2, g2, be2, W3, b3)
    return out_p.reshape(E, D, 1)
